# baseline (speedup 1.0000x reference)
import jax
import jax.numpy as jnp
from jax.experimental import pallas as pl

B, V, C = 64, 32, 64
R = B * V
CHUNK = 256
NCH = R // CHUNK


def _vae_kernel(x_ref, u_ref, iw_ref, s_ref):
    x = x_ref[:]
    u = u_ref[:]
    tiled = jnp.broadcast_to(x[:, None, :], (CHUNK, C - 1, C))
    ii = jax.lax.broadcasted_iota(jnp.int32, (CHUNK, C - 1, C), 1)
    jj = jax.lax.broadcasted_iota(jnp.int32, (CHUNK, C - 1, C), 2)
    dl = jnp.where(jj <= ii, -jnp.inf, tiled)
    amax = jnp.max(dl, axis=-1)
    ssum = jnp.sum(jnp.exp(dl - amax[:, :, None]), axis=-1)
    denom = jnp.log(ssum) + amax
    sb = x[:, : C - 1] - denom

    sg = jax.nn.sigmoid(sb)
    sg_abs = jax.nn.sigmoid(jnp.abs(sb))
    cond = sb >= -1e-5
    safe_den = jnp.where(cond, 1.0, 1.0 - 2.0 * sg)
    bzm = jnp.where(cond, jnp.zeros_like(sb), (1.0 - sg) ** 2 / safe_den)
    for k in (1, 2, 4, 8, 16, 32):
        shifted = jnp.concatenate(
            [jnp.ones((CHUNK, k), jnp.float32), bzm[:, : C - 1 - k]], axis=1)
        bzm = bzm * shifted
    iw = jnp.concatenate(
        [sg_abs[:, 0:1], bzm[:, :-1] * sg_abs[:, 1:], bzm[:, -1:]], axis=-1)
    iw_ref[:] = iw

    hit = u[:, : C - 1] < sg
    lane = jax.lax.broadcasted_iota(jnp.int32, (CHUNK, C - 1), 1)
    s_ref[:] = jnp.min(
        jnp.where(hit, lane, C - 1), axis=-1, keepdims=True).astype(jnp.int32)


@jax.jit
def kernel(encoder_logits, u_noise):
    x = encoder_logits.reshape(R, C)
    u = u_noise.reshape(R, C)
    iw, samp = pl.pallas_call(
        _vae_kernel,
        grid=(NCH,),
        in_specs=[
            pl.BlockSpec((CHUNK, C), lambda i: (i, 0)),
            pl.BlockSpec((CHUNK, C), lambda i: (i, 0)),
        ],
        out_specs=(
            pl.BlockSpec((CHUNK, C), lambda i: (i, 0)),
            pl.BlockSpec((CHUNK, 1), lambda i: (i, 0)),
        ),
        out_shape=(
            jax.ShapeDtypeStruct((R, C), jnp.float32),
            jax.ShapeDtypeStruct((R, 1), jnp.int32),
        ),
    )(x, u)
    return iw.reshape(B, V, C), samp.reshape(B, V)
